# baseline (speedup 1.0000x reference)
import jax
import jax.numpy as jnp
from jax import lax
from jax.experimental import pallas as pl
from jax.experimental.pallas import tpu as pltpu

_HIDDEN = 2048
_EXPERTS = 64
_TC = 1024
_NBUF = 3


def _router_logits_kernel(x_hbm, w_ref, o_hbm, xbufs, obufs, in_sems, out_sems):
    nchunks = x_hbm.shape[0] // _TC

    def in_copy(j):
        return pltpu.make_async_copy(
            x_hbm.at[pl.ds(j * _TC, _TC), :], xbufs.at[j % _NBUF], in_sems.at[j % _NBUF]
        )

    def out_copy(j):
        return pltpu.make_async_copy(
            obufs.at[j % _NBUF], o_hbm.at[pl.ds(j * _TC, _TC), :], out_sems.at[j % _NBUF]
        )

    for j in range(_NBUF):
        in_copy(j).start()

    for j in range(nchunks):
        in_copy(j).wait()
        if j >= _NBUF:
            out_copy(j - _NBUF).wait()
        obufs[j % _NBUF, :, :] = lax.dot_general(
            xbufs[j % _NBUF],
            w_ref[...],
            dimension_numbers=(((1,), (1,)), ((), ())),
            preferred_element_type=jnp.float32,
        )
        out_copy(j).start()
        if j + _NBUF < nchunks:
            in_copy(j + _NBUF).start()

    for j in range(nchunks - _NBUF, nchunks):
        out_copy(j).wait()


def kernel(hidden_states, weight):
    hs = hidden_states.reshape(-1, _HIDDEN)
    m = hs.shape[0]
    out = pl.pallas_call(
        _router_logits_kernel,
        in_specs=[
            pl.BlockSpec(memory_space=pltpu.MemorySpace.HBM),
            pl.BlockSpec(memory_space=pltpu.MemorySpace.VMEM),
        ],
        out_specs=pl.BlockSpec(memory_space=pltpu.MemorySpace.HBM),
        out_shape=jax.ShapeDtypeStruct((m, _EXPERTS), jnp.float32),
        scratch_shapes=[
            pltpu.VMEM((_NBUF, _TC, _HIDDEN), jnp.float32),
            pltpu.VMEM((_NBUF, _TC, _EXPERTS), jnp.float32),
            pltpu.SemaphoreType.DMA((_NBUF,)),
            pltpu.SemaphoreType.DMA((_NBUF,)),
        ],
    )(hs, weight)
    return out


# speedup vs baseline: 1.0489x; 1.0489x over previous
import jax
import jax.numpy as jnp
from jax import lax
from jax.experimental import pallas as pl
from jax.experimental.pallas import tpu as pltpu

_HIDDEN = 2048
_EXPERTS = 64
_TM = 1024


def _router_logits_kernel(x_ref, w_ref, o_ref):
    o_ref[...] = lax.dot_general(
        x_ref[...].astype(jnp.bfloat16),
        w_ref[...].astype(jnp.bfloat16),
        dimension_numbers=(((1,), (1,)), ((), ())),
        preferred_element_type=jnp.float32,
    )


def kernel(hidden_states, weight):
    hs = hidden_states.reshape(-1, _HIDDEN)
    m = hs.shape[0]
    grid = (m // _TM,)
    out = pl.pallas_call(
        _router_logits_kernel,
        grid=grid,
        in_specs=[
            pl.BlockSpec((_TM, _HIDDEN), lambda i: (i, 0)),
            pl.BlockSpec((_EXPERTS, _HIDDEN), lambda i: (0, 0)),
        ],
        out_specs=pl.BlockSpec((_TM, _EXPERTS), lambda i: (i, 0)),
        out_shape=jax.ShapeDtypeStruct((m, _EXPERTS), jnp.float32),
        compiler_params=pltpu.CompilerParams(
            dimension_semantics=("arbitrary",),
        ),
    )(hs, weight)
    return out
